# Initial kernel scaffold; baseline (speedup 1.0000x reference)
import jax
import jax.numpy as jnp
from jax.experimental import pallas as pl


def kernel(x, embedding, fc_w, fc_b):
    raise NotImplementedError("write your pallas kernel here")


# speedup vs baseline: 5.9035x; 5.9035x over previous
import functools

import jax
import jax.numpy as jnp
import numpy as np
from jax import lax
from jax.experimental import pallas as pl
from jax.experimental.pallas import tpu as pltpu
from jax.experimental.pallas import tpu_sc as plsc

_VOCAB = 8
_DIM = 16
_OUT = 8
_N = 16384 * 200
_NW = 32
_PER_W = _N // _NW
_CHUNK = 6400
_NCH = _PER_W // _CHUNK
_L = 16

_mesh = plsc.VectorSubcoreMesh(core_axis_name="c", subcore_axis_name="s")


@functools.partial(
    pl.kernel,
    out_type=jax.ShapeDtypeStruct((_N, _OUT), jnp.float32),
    mesh=_mesh,
    compiler_params=pltpu.CompilerParams(
        needs_layout_passes=False, use_tc_tiling_on_sc=False),
    scratch_types=[
        pltpu.VMEM((_VOCAB, _DIM), jnp.float32),
        pltpu.VMEM((_VOCAB, _DIM), jnp.float32),
        pltpu.VMEM((_L,), jnp.float32),
        pltpu.VMEM((_OUT, _VOCAB), jnp.float32),
        pltpu.VMEM((_CHUNK,), jnp.int32),
        pltpu.VMEM((_CHUNK, _OUT), jnp.float32),
    ],
)
def _sc_lookup(emb_hbm, fcw_hbm, fcb_hbm, x_hbm, out_hbm,
               emb_v, fcw_v, fcb_v, tab_v, idx_v, rows_v):
    wid = lax.axis_index("s") * 2 + lax.axis_index("c")
    pltpu.sync_copy(emb_hbm, emb_v)
    pltpu.sync_copy(fcw_hbm, fcw_v)
    pltpu.sync_copy(fcb_hbm, fcb_v)

    iota = lax.iota(jnp.int32, _L)

    def splat(val):
        return jnp.broadcast_to(jnp.int32(val), (_L,))

    for j in range(4):
        p = iota + splat(j * _L)
        v_idx = lax.shift_right_logical(p, splat(3))
        k_idx = jnp.bitwise_and(p, splat(7))
        acc = plsc.load_gather(fcb_v, [v_idx])
        for d in range(_DIM):
            dd = splat(d)
            wv = plsc.load_gather(fcw_v, [v_idx, dd])
            ek = plsc.load_gather(emb_v, [k_idx, dd])
            acc = acc + wv * ek
        plsc.store_scatter(tab_v, [v_idx, k_idx], acc)

    base_w = wid * _PER_W

    def chunk_body(g, carry):
        base = base_w + g * _CHUNK
        pltpu.sync_copy(x_hbm.at[pl.ds(base, _CHUNK)], idx_v)

        def tok_body(i, c):
            tok = idx_v[pl.ds(i * _L, _L)]
            pos = iota + splat(i * _L)
            for v in range(_OUT):
                r = plsc.load_gather(tab_v, [splat(v), tok])
                plsc.store_scatter(rows_v, [pos, splat(v)], r)
            return c

        lax.fori_loop(0, _CHUNK // _L, tok_body, 0)
        pltpu.sync_copy(rows_v, out_hbm.at[pl.ds(base, _CHUNK)])
        return carry

    lax.fori_loop(0, _NCH, chunk_body, 0)


def kernel(x, embedding, fc_w, fc_b):
    x_flat = x.reshape(-1)
    fcb_pad = jnp.pad(fc_b, (0, _L - _VOCAB))
    out = _sc_lookup(embedding, fc_w, fcb_pad, x_flat)
    return out.reshape(x.shape[0], x.shape[1], _OUT)


# speedup vs baseline: 6.4084x; 1.0855x over previous
import functools

import jax
import jax.numpy as jnp
import numpy as np
from jax import lax
from jax.experimental import pallas as pl
from jax.experimental.pallas import tpu as pltpu
from jax.experimental.pallas import tpu_sc as plsc

_VOCAB = 8
_DIM = 16
_OUT = 8
_N = 16384 * 200
_NW = 32
_PER_W = _N // _NW
_CHUNK = 6400
_NCH = _PER_W // _CHUNK
_L = 16

_mesh = plsc.VectorSubcoreMesh(core_axis_name="c", subcore_axis_name="s")


@functools.partial(
    pl.kernel,
    out_type=jax.ShapeDtypeStruct((_N, _OUT), jnp.float32),
    mesh=_mesh,
    compiler_params=pltpu.CompilerParams(
        needs_layout_passes=False, use_tc_tiling_on_sc=False),
    scratch_types=[
        pltpu.VMEM((_VOCAB, _DIM), jnp.float32),
        pltpu.VMEM((_VOCAB, _DIM), jnp.float32),
        pltpu.VMEM((_L,), jnp.float32),
        pltpu.VMEM((_OUT, _VOCAB), jnp.float32),
        pltpu.VMEM((_CHUNK,), jnp.int32),
        pltpu.VMEM((_CHUNK, _OUT), jnp.float32),
    ],
)
def _sc_lookup(emb_hbm, fcw_hbm, fcb_hbm, x_hbm, out_hbm,
               emb_v, fcw_v, fcb_v, tab_v, idx_v, rows_v):
    wid = lax.axis_index("s") * 2 + lax.axis_index("c")
    pltpu.sync_copy(emb_hbm, emb_v)
    pltpu.sync_copy(fcw_hbm, fcw_v)
    pltpu.sync_copy(fcb_hbm, fcb_v)

    iota = lax.iota(jnp.int32, _L)

    def splat(val):
        return jnp.broadcast_to(jnp.int32(val), (_L,))

    for j in range(4):
        p = iota + splat(j * _L)
        v_idx = lax.shift_right_logical(p, splat(3))
        k_idx = jnp.bitwise_and(p, splat(7))
        acc = plsc.load_gather(fcb_v, [v_idx])
        for d in range(_DIM):
            dd = splat(d)
            wv = plsc.load_gather(fcw_v, [v_idx, dd])
            ek = plsc.load_gather(emb_v, [k_idx, dd])
            acc = acc + wv * ek
        plsc.store_scatter(tab_v, [v_idx, k_idx], acc)

    base_w = wid * _PER_W

    def chunk_body(g, carry):
        base = base_w + g * _CHUNK
        pltpu.sync_copy(x_hbm.at[pl.ds(base, _CHUNK)], idx_v)

        @plsc.parallel_loop(0, _CHUNK, step=_L, unroll=4)
        def _tok_body(i):
            tok = idx_v[pl.ds(i, _L)]
            pos = iota + splat(i)
            for v in range(_OUT):
                r = plsc.load_gather(tab_v, [splat(v), tok])
                plsc.store_scatter(rows_v, [pos, splat(v)], r)
        pltpu.sync_copy(rows_v, out_hbm.at[pl.ds(base, _CHUNK)])
        return carry

    lax.fori_loop(0, _NCH, chunk_body, 0)


def kernel(x, embedding, fc_w, fc_b):
    x_flat = x.reshape(-1)
    fcb_pad = jnp.pad(fc_b, (0, _L - _VOCAB))
    out = _sc_lookup(embedding, fc_w, fcb_pad, x_flat)
    return out.reshape(x.shape[0], x.shape[1], _OUT)


# speedup vs baseline: 100.3963x; 15.6665x over previous
import functools

import jax
import jax.numpy as jnp
from jax import lax
from jax.experimental import pallas as pl
from jax.experimental.pallas import tpu as pltpu
from jax.experimental.pallas import tpu_sc as plsc

_VOCAB = 8
_DIM = 16
_OUT = 8
_B = 16384
_SEQ = 200
_N = _B * _SEQ
_NW = 32
_PER_W = _N // _NW
_CHUNK = 4096
_NCH = _PER_W // _CHUNK
_L = 16

_mesh = plsc.VectorSubcoreMesh(core_axis_name="c", subcore_axis_name="s")


@functools.partial(
    pl.kernel,
    out_type=jax.ShapeDtypeStruct((_N * _OUT,), jnp.float32),
    mesh=_mesh,
    compiler_params=pltpu.CompilerParams(
        needs_layout_passes=False, use_tc_tiling_on_sc=False),
    scratch_types=[
        pltpu.VMEM((_VOCAB, _DIM), jnp.float32),
        pltpu.VMEM((_VOCAB, _DIM), jnp.float32),
        pltpu.VMEM((_L,), jnp.float32),
        pltpu.VMEM((_OUT, _VOCAB), jnp.float32),
        pltpu.VMEM((_CHUNK,), jnp.int32),
        pltpu.VMEM((_CHUNK * _OUT,), jnp.float32),
    ],
)
def _sc_lookup(emb_hbm, fcw_hbm, fcb_hbm, xt_hbm, out_hbm,
               emb_v, fcw_v, fcb_v, tab_v, idx_v, rows_v):
    wid = lax.axis_index("s") * 2 + lax.axis_index("c")
    pltpu.sync_copy(emb_hbm, emb_v)
    pltpu.sync_copy(fcw_hbm, fcw_v)
    pltpu.sync_copy(fcb_hbm, fcb_v)

    iota = lax.iota(jnp.int32, _L)

    def splat(val):
        return jnp.broadcast_to(jnp.int32(val), (_L,))

    for j in range(4):
        p = iota + splat(j * _L)
        v_idx = lax.shift_right_logical(p, splat(3))
        k_idx = jnp.bitwise_and(p, splat(7))
        acc = plsc.load_gather(fcb_v, [v_idx])
        for d in range(_DIM):
            dd = splat(d)
            wv = plsc.load_gather(fcw_v, [v_idx, dd])
            ek = plsc.load_gather(emb_v, [k_idx, dd])
            acc = acc + wv * ek
        plsc.store_scatter(tab_v, [v_idx, k_idx], acc)

    base_w = wid * _PER_W

    def chunk_body(g, carry):
        base = base_w + g * _CHUNK
        pltpu.sync_copy(xt_hbm.at[pl.ds(base, _CHUNK)], idx_v)

        @plsc.parallel_loop(0, _CHUNK, step=_L, unroll=4)
        def _tok_body(i):
            tok = idx_v[pl.ds(i, _L)]
            pos = ((i >> 7) << 10) | (i & 127)
            for v in range(_OUT):
                r = plsc.load_gather(tab_v, [splat(v), tok])
                rows_v[pl.ds(pos + v * 128, _L)] = r

        pltpu.sync_copy(rows_v, out_hbm.at[pl.ds(base * _OUT, _CHUNK * _OUT)])
        return carry

    lax.fori_loop(0, _NCH, chunk_body, 0)


def kernel(x, embedding, fc_w, fc_b):
    xt_flat = x.T.reshape(-1)
    fcb_pad = jnp.pad(fc_b, (0, _L - _VOCAB))
    flat = _sc_lookup(embedding, fc_w, fcb_pad, xt_flat)
    r4 = flat.reshape(_SEQ, _B // 128, _OUT, 128)
    return r4.transpose(1, 3, 0, 2).reshape(_B, _SEQ, _OUT)


# speedup vs baseline: 153.8542x; 1.5325x over previous
import functools

import jax
import jax.numpy as jnp
from jax import lax
from jax.experimental import pallas as pl
from jax.experimental.pallas import tpu as pltpu
from jax.experimental.pallas import tpu_sc as plsc

_VOCAB = 8
_DIM = 16
_OUT = 8
_B = 16384
_SEQ = 200
_N = _B * _SEQ
_NW = 32
_PER_W = _N // _NW
_CHUNK = 6400
_NCH = _PER_W // _CHUNK
_L = 16

_mesh = plsc.VectorSubcoreMesh(core_axis_name="c", subcore_axis_name="s")


@functools.partial(
    pl.kernel,
    out_type=jax.ShapeDtypeStruct((_N * _OUT,), jnp.float32),
    mesh=_mesh,
    compiler_params=pltpu.CompilerParams(
        needs_layout_passes=False, use_tc_tiling_on_sc=False),
    scratch_types=[
        pltpu.VMEM((_VOCAB, _DIM), jnp.float32),
        pltpu.VMEM((_VOCAB, _DIM), jnp.float32),
        pltpu.VMEM((_L,), jnp.float32),
        pltpu.VMEM((_OUT, _VOCAB), jnp.float32),
        pltpu.VMEM((_CHUNK,), jnp.int32),
        pltpu.VMEM((_CHUNK,), jnp.int32),
        pltpu.VMEM((_CHUNK * _OUT,), jnp.float32),
        pltpu.VMEM((_CHUNK * _OUT,), jnp.float32),
        pltpu.SemaphoreType.DMA,
        pltpu.SemaphoreType.DMA,
        pltpu.SemaphoreType.DMA,
        pltpu.SemaphoreType.DMA,
    ],
)
def _sc_lookup(emb_hbm, fcw_hbm, fcb_hbm, xt_hbm, out_hbm,
               emb_v, fcw_v, fcb_v, tab_v,
               idx0, idx1, rows0, rows1, sin0, sin1, sout0, sout1):
    idx = (idx0, idx1)
    rows = (rows0, rows1)
    sin = (sin0, sin1)
    sout = (sout0, sout1)
    wid = lax.axis_index("s") * 2 + lax.axis_index("c")
    pltpu.sync_copy(emb_hbm, emb_v)
    pltpu.sync_copy(fcw_hbm, fcw_v)
    pltpu.sync_copy(fcb_hbm, fcb_v)

    iota = lax.iota(jnp.int32, _L)

    def splat(val):
        return jnp.broadcast_to(jnp.int32(val), (_L,))

    for j in range(4):
        p = iota + splat(j * _L)
        v_idx = lax.shift_right_logical(p, splat(3))
        k_idx = jnp.bitwise_and(p, splat(7))
        acc = plsc.load_gather(fcb_v, [v_idx])
        for d in range(_DIM):
            dd = splat(d)
            wv = plsc.load_gather(fcw_v, [v_idx, dd])
            ek = plsc.load_gather(emb_v, [k_idx, dd])
            acc = acc + wv * ek
        plsc.store_scatter(tab_v, [v_idx, k_idx], acc)

    base_w = wid * _PER_W

    def in_src(g):
        return xt_hbm.at[pl.ds(base_w + g * _CHUNK, _CHUNK)]

    def out_dst(g):
        return out_hbm.at[pl.ds((base_w + g * _CHUNK) * _OUT, _CHUNK * _OUT)]

    for b in range(2):
        pltpu.async_copy(in_src(b), idx[b], sin[b])

    def pair_body(p, carry):
        for b in range(2):
            g = p * 2 + b
            pltpu.make_async_copy(in_src(g), idx[b], sin[b]).wait()

            @pl.when(p >= 1)
            def _wait_out():
                pltpu.make_async_copy(rows[b], out_dst(g), sout[b]).wait()

            @plsc.parallel_loop(0, _CHUNK, step=_L, unroll=4)
            def _tok_body(i):
                tok = idx[b][pl.ds(i, _L)]
                pos = ((i >> 7) << 10) | (i & 127)
                for v in range(_OUT):
                    r = plsc.load_gather(tab_v, [splat(v), tok])
                    rows[b][pl.ds(pos + v * 128, _L)] = r

            pltpu.async_copy(rows[b], out_dst(g), sout[b])

            @pl.when(g + 2 < _NCH)
            def _next_in():
                pltpu.async_copy(in_src(g + 2), idx[b], sin[b])
        return carry

    lax.fori_loop(0, _NCH // 2, pair_body, 0)

    for b in range(2):
        pltpu.make_async_copy(rows[b], out_dst(0), sout[b]).wait()


def kernel(x, embedding, fc_w, fc_b):
    xt_flat = x.T.reshape(-1)
    fcb_pad = jnp.pad(fc_b, (0, _L - _VOCAB))
    flat = _sc_lookup(embedding, fc_w, fcb_pad, xt_flat)
    r4 = flat.reshape(_SEQ, _B // 128, _OUT, 128)
    return r4.transpose(1, 3, 0, 2).reshape(_B, _SEQ, _OUT)


# speedup vs baseline: 174.6482x; 1.1352x over previous
import functools

import jax
import jax.numpy as jnp
from jax import lax
from jax.experimental import pallas as pl
from jax.experimental.pallas import tpu as pltpu
from jax.experimental.pallas import tpu_sc as plsc

_VOCAB = 8
_DIM = 16
_OUT = 8
_B = 16384
_SEQ = 200
_N = _B * _SEQ
_NW = 32
_PER_W = _N // _NW
_K = 2
_CHUNK = _K * 1024
_NCH = _PER_W // _CHUNK
_CPL = 128 // _K
_L = 16
_LSTRIDE = _B * _OUT

_mesh = plsc.VectorSubcoreMesh(core_axis_name="c", subcore_axis_name="s")


@functools.partial(
    pl.kernel,
    out_type=jax.ShapeDtypeStruct((_N * _OUT,), jnp.float32),
    mesh=_mesh,
    compiler_params=pltpu.CompilerParams(
        needs_layout_passes=False, use_tc_tiling_on_sc=False),
    scratch_types=[
        pltpu.VMEM((_VOCAB, _DIM), jnp.float32),
        pltpu.VMEM((_VOCAB, _DIM), jnp.float32),
        pltpu.VMEM((_L,), jnp.float32),
        pltpu.VMEM((_OUT, _VOCAB), jnp.float32),
        pltpu.VMEM((_CHUNK,), jnp.int32),
        pltpu.VMEM((_CHUNK,), jnp.int32),
        pltpu.VMEM((_CHUNK * _OUT,), jnp.float32),
        pltpu.VMEM((_CHUNK * _OUT,), jnp.float32),
        pltpu.SemaphoreType.DMA,
        pltpu.SemaphoreType.DMA,
        pltpu.SemaphoreType.DMA,
        pltpu.SemaphoreType.DMA,
    ],
)
def _sc_lookup(emb_hbm, fcw_hbm, fcb_hbm, xt_hbm, out_hbm,
               emb_v, fcw_v, fcb_v, tab_v,
               idx0, idx1, rows0, rows1, sin0, sin1, sout0, sout1):
    idx = (idx0, idx1)
    rows = (rows0, rows1)
    sin = (sin0, sin1)
    sout = (sout0, sout1)
    wid = lax.axis_index("s") * 2 + lax.axis_index("c")
    pltpu.sync_copy(emb_hbm, emb_v)
    pltpu.sync_copy(fcw_hbm, fcw_v)
    pltpu.sync_copy(fcb_hbm, fcb_v)

    iota = lax.iota(jnp.int32, _L)

    def splat(val):
        return jnp.broadcast_to(jnp.int32(val), (_L,))

    for j in range(4):
        p = iota + splat(j * _L)
        v_idx = lax.shift_right_logical(p, splat(3))
        k_idx = jnp.bitwise_and(p, splat(7))
        acc = plsc.load_gather(fcb_v, [v_idx])
        for d in range(_DIM):
            dd = splat(d)
            wv = plsc.load_gather(fcw_v, [v_idx, dd])
            ek = plsc.load_gather(emb_v, [k_idx, dd])
            acc = acc + wv * ek
        plsc.store_scatter(tab_v, [v_idx, k_idx], acc)

    base_w = wid * _NCH

    def in_src(g):
        return xt_hbm.at[pl.ds((base_w + g) * _CHUNK, _CHUNK)]

    def start_out(g, b):
        u = base_w + g
        l_hi = u // _CPL
        bh0 = (u % _CPL) * _K
        base = l_hi * 8 * _LSTRIDE + bh0 * 1024
        for l_lo in range(8):
            pltpu.async_copy(
                rows[b].at[pl.ds(l_lo * (_K * 1024), _K * 1024)],
                out_hbm.at[pl.ds(base + l_lo * _LSTRIDE, _K * 1024)],
                sout[b])

    def drain_out(b):
        pltpu.make_async_copy(
            rows[b], out_hbm.at[pl.ds(0, _CHUNK * _OUT)], sout[b]).wait()

    for b in range(2):
        pltpu.async_copy(in_src(b), idx[b], sin[b])

    def pair_body(p, carry):
        for b in range(2):
            g = p * 2 + b
            pltpu.make_async_copy(in_src(g), idx[b], sin[b]).wait()

            @pl.when(p >= 1)
            def _wait_out():
                drain_out(b)

            @plsc.parallel_loop(0, _CHUNK, step=_L, unroll=4)
            def _tok_body(i):
                tok = idx[b][pl.ds(i, _L)]
                pos = (((i >> 7) & 7) * (_K * 1024)) | ((i >> 10) << 10) | (i & 127)
                for v in range(_OUT):
                    r = plsc.load_gather(tab_v, [splat(v), tok])
                    rows[b][pl.ds(pos + v * 128, _L)] = r

            start_out(g, b)

            @pl.when(g + 2 < _NCH)
            def _next_in():
                pltpu.async_copy(in_src(g + 2), idx[b], sin[b])
        return carry

    lax.fori_loop(0, _NCH // 2, pair_body, 0)

    for b in range(2):
        drain_out(b)


def kernel(x, embedding, fc_w, fc_b):
    x_feed = x.reshape(128, 128, 25, 8).transpose(2, 0, 3, 1).reshape(-1)
    fcb_pad = jnp.pad(fc_b, (0, _L - _VOCAB))
    flat = _sc_lookup(embedding, fc_w, fcb_pad, x_feed)
    r4 = flat.reshape(_SEQ, _B // 128, _OUT, 128)
    return r4.transpose(1, 3, 0, 2).reshape(_B, _SEQ, _OUT)


# speedup vs baseline: 176.2613x; 1.0092x over previous
import functools

import jax
import jax.numpy as jnp
from jax import lax
from jax.experimental import pallas as pl
from jax.experimental.pallas import tpu as pltpu
from jax.experimental.pallas import tpu_sc as plsc

_VOCAB = 8
_DIM = 16
_OUT = 8
_B = 16384
_SEQ = 200
_N = _B * _SEQ
_NW = 32
_PER_W = _N // _NW
_K = 2
_CHUNK = _K * 1024
_NCH = _PER_W // _CHUNK
_CPL = 128 // _K
_L = 16
_LSTRIDE = _B * _OUT

_mesh = plsc.VectorSubcoreMesh(core_axis_name="c", subcore_axis_name="s")


@functools.partial(
    pl.kernel,
    out_type=jax.ShapeDtypeStruct((_N * _OUT,), jnp.float32),
    mesh=_mesh,
    compiler_params=pltpu.CompilerParams(
        needs_layout_passes=False, use_tc_tiling_on_sc=False),
    scratch_types=[
        pltpu.VMEM((_VOCAB, _DIM), jnp.float32),
        pltpu.VMEM((_VOCAB, _DIM), jnp.float32),
        pltpu.VMEM((_L,), jnp.float32),
        pltpu.VMEM((_OUT, _VOCAB), jnp.float32),
        pltpu.VMEM((_CHUNK,), jnp.int32),
        pltpu.VMEM((_CHUNK,), jnp.int32),
        pltpu.VMEM((_CHUNK * _OUT,), jnp.float32),
        pltpu.VMEM((_CHUNK * _OUT,), jnp.float32),
        pltpu.SemaphoreType.DMA,
        pltpu.SemaphoreType.DMA,
        pltpu.SemaphoreType.DMA,
        pltpu.SemaphoreType.DMA,
    ],
)
def _sc_lookup(emb_hbm, fcw_hbm, fcb_hbm, xt_hbm, out_hbm,
               emb_v, fcw_v, fcb_v, tab_v,
               idx0, idx1, rows0, rows1, sin0, sin1, sout0, sout1):
    idx = (idx0, idx1)
    rows = (rows0, rows1)
    sin = (sin0, sin1)
    sout = (sout0, sout1)
    wid = lax.axis_index("s") * 2 + lax.axis_index("c")
    base_w = wid * _NCH

    def in_src(g):
        return xt_hbm.at[pl.ds((base_w + g) * _CHUNK, _CHUNK)]

    for b in range(2):
        pltpu.async_copy(in_src(b), idx[b], sin[b])

    pltpu.sync_copy(emb_hbm, emb_v)
    pltpu.sync_copy(fcw_hbm, fcw_v)
    pltpu.sync_copy(fcb_hbm, fcb_v)

    iota = lax.iota(jnp.int32, _L)

    def splat(val):
        return jnp.broadcast_to(jnp.int32(val), (_L,))

    for j in range(4):
        p = iota + splat(j * _L)
        v_idx = lax.shift_right_logical(p, splat(3))
        k_idx = jnp.bitwise_and(p, splat(7))
        acc = plsc.load_gather(fcb_v, [v_idx])
        for d in range(_DIM):
            dd = splat(d)
            wv = plsc.load_gather(fcw_v, [v_idx, dd])
            ek = plsc.load_gather(emb_v, [k_idx, dd])
            acc = acc + wv * ek
        plsc.store_scatter(tab_v, [v_idx, k_idx], acc)

    def start_out(g, b):
        u = base_w + g
        l_hi = u // _CPL
        bh0 = (u % _CPL) * _K
        base = l_hi * 8 * _LSTRIDE + bh0 * 1024
        for l_lo in range(8):
            pltpu.async_copy(
                rows[b].at[pl.ds(l_lo * (_K * 1024), _K * 1024)],
                out_hbm.at[pl.ds(base + l_lo * _LSTRIDE, _K * 1024)],
                sout[b])

    def drain_out(b):
        pltpu.make_async_copy(
            rows[b], out_hbm.at[pl.ds(0, _CHUNK * _OUT)], sout[b]).wait()

    def pair_body(p, carry):
        for b in range(2):
            g = p * 2 + b
            pltpu.make_async_copy(in_src(g), idx[b], sin[b]).wait()

            @pl.when(p >= 1)
            def _wait_out():
                drain_out(b)

            @plsc.parallel_loop(0, _CHUNK, step=_L, unroll=8)
            def _tok_body(i):
                tok = idx[b][pl.ds(i, _L)]
                pos = (((i >> 7) & 7) * (_K * 1024)) | ((i >> 10) << 10) | (i & 127)
                for v in range(_OUT):
                    r = plsc.load_gather(tab_v, [splat(v), tok])
                    rows[b][pl.ds(pos + v * 128, _L)] = r

            start_out(g, b)

            @pl.when(g + 2 < _NCH)
            def _next_in():
                pltpu.async_copy(in_src(g + 2), idx[b], sin[b])
        return carry

    lax.fori_loop(0, _NCH // 2, pair_body, 0)

    for b in range(2):
        drain_out(b)


def kernel(x, embedding, fc_w, fc_b):
    x_feed = x.reshape(128, 128, 25, 8).transpose(2, 0, 3, 1).reshape(-1)
    fcb_pad = jnp.pad(fc_b, (0, _L - _VOCAB))
    flat = _sc_lookup(embedding, fc_w, fcb_pad, x_feed)
    r4 = flat.reshape(_SEQ, _B // 128, _OUT, 128)
    return r4.transpose(1, 3, 0, 2).reshape(_B, _SEQ, _OUT)
